# baseline (speedup 1.0000x reference)
import functools

import jax
import jax.numpy as jnp
from jax import lax
from jax.experimental import pallas as pl
from jax.experimental.pallas import tpu as pltpu
from jax.experimental.pallas import tpu_sc as plsc

_NC = 2
_NS = 16
_NW = _NC * _NS
_L = 16
_NSLOT = 3
_IDEPTH = 6


def _make_agg(n, d, nb, k):
  rows_per_tile = -(-n // (_NS * k)) * k
  n_pad = rows_per_tile * _NS
  nz = rows_per_tile // k
  mesh = plsc.VectorSubcoreMesh(core_axis_name="c", subcore_axis_name="s")

  @functools.partial(
      pl.kernel,
      out_type=jax.ShapeDtypeStruct((_NC, n_pad, d), jnp.float32),
      mesh=mesh,
      scratch_types=[
          pltpu.VMEM((_IDEPTH, k), jnp.int32),
          pltpu.VMEM((_IDEPTH, k), jnp.int32),
          pltpu.VMEM((_IDEPTH, k), jnp.float32),
          pltpu.VMEM((k, d), jnp.float32),
          pltpu.VMEM((k, d), jnp.float32),
          pltpu.VMEM((k, d), jnp.float32),
          pltpu.VMEM_SHARED((n_pad, d), jnp.float32),
          pltpu.SemaphoreType.DMA,
          pltpu.SemaphoreType.DMA,
          pltpu.SemaphoreType.DMA,
          pltpu.SemaphoreType.DMA,
          pltpu.SemaphoreType.DMA,
          pltpu.SemaphoreType.DMA,
          pltpu.SemaphoreType.DMA,
          pltpu.SemaphoreType.DMA,
          pltpu.SemaphoreType.DMA,
          pltpu.SemaphoreType.DMA,
          pltpu.SemaphoreType.DMA,
          pltpu.SemaphoreType.DMA,
      ],
  )
  def agg(x_hbm, row_hbm, col_hbm, w_hbm, out_hbm,
          colq, rowq, wq, r0, r1, r2, acc,
          g0, g1, g2, s0, s1, s2, i0, i1, i2, i3, i4, i5):
    c = lax.axis_index("c")
    s = lax.axis_index("s")
    wid = c * _NS + s
    bufs = (r0, r1, r2)
    gsems = (g0, g1, g2)
    ssems = (s0, s1, s2)
    isems = (i0, i1, i2, i3, i4, i5)

    def start_idx(b, q):
      pltpu.async_copy(col_hbm.at[wid, b], colq.at[q], isems[q])
      pltpu.async_copy(row_hbm.at[wid, b], rowq.at[q], isems[q])
      pltpu.async_copy(w_hbm.at[wid, b], wq.at[q], isems[q])

    def wait_idx(b, q):
      pltpu.make_async_copy(col_hbm.at[wid, b], colq.at[q], isems[q]).wait()
      pltpu.make_async_copy(row_hbm.at[wid, b], rowq.at[q], isems[q]).wait()
      pltpu.make_async_copy(w_hbm.at[wid, b], wq.at[q], isems[q]).wait()

    def start_gather(q, j):
      pltpu.async_copy(x_hbm.at[colq.at[q]], bufs[j], gsems[j])

    def wait_gather(q, j):
      pltpu.make_async_copy(x_hbm.at[colq.at[q]], bufs[j], gsems[j]).wait()

    def start_scatter(q, j):
      pltpu.async_copy(bufs[j], acc.at[rowq.at[q]], ssems[j], add=True)

    def wait_scatter(q, j):
      pltpu.make_async_copy(bufs[j], acc.at[rowq.at[q]], ssems[j]).wait()

    def scale(q, j):
      buf = bufs[j]

      def scale16(g, _):
        wvec = wq[q, pl.ds(g * _L, _L)]
        for j16 in range(_L):
          w = wvec[j16]
          jj = g * _L + j16
          for t in range(d // _L):
            buf[jj, pl.ds(t * _L, _L)] = buf[jj, pl.ds(t * _L, _L)] * w
        return 0

      lax.fori_loop(0, k // _L, scale16, 0)

    zero = jnp.zeros((_L,), jnp.float32)

    def zstore(i, _):
      r = i // (d // _L)
      col0 = (i % (d // _L)) * _L
      r0[r, pl.ds(col0, _L)] = zero
      return 0

    lax.fori_loop(0, k * (d // _L), zstore, 0)
    for t in range(nz):
      pltpu.sync_copy(r0, acc.at[pl.ds(s * rows_per_tile + t * k, k)])
    plsc.subcore_barrier()

    for q in range(_IDEPTH):
      start_idx(q, q)
    for j in range(_NSLOT):
      wait_idx(j, j)
      start_gather(j, j)

    def body(i, _):
      for jj in range(_IDEPTH):
        b = i * _IDEPTH + jj
        j = jj % _NSLOT
        wait_gather(jj, j)
        scale(jj, j)
        start_scatter(jj, j)
        bp = b - 1
        jp = (jj + _IDEPTH - 1) % _IDEPTH
        rp = jp % _NSLOT

        @pl.when(jnp.logical_and(bp >= 0, bp + _NSLOT < nb))
        def _():
          wait_scatter(jp, rp)
          wait_idx(bp + _NSLOT, (jp + _NSLOT) % _IDEPTH)
          start_gather((jp + _NSLOT) % _IDEPTH, rp)

        @pl.when(jnp.logical_and(bp >= 0, bp + _IDEPTH < nb))
        def _():
          start_idx(bp + _IDEPTH, jp)

      return 0

    lax.fori_loop(0, nb // _IDEPTH, body, 0)

    for j in range(_NSLOT):
      q = (nb - _NSLOT + j) % _IDEPTH
      wait_scatter(q, q % _NSLOT)

    plsc.subcore_barrier()
    pltpu.sync_copy(acc.at[pl.ds(s * rows_per_tile, rows_per_tile)],
                    out_hbm.at[c, pl.ds(s * rows_per_tile, rows_per_tile)])

  return agg


def _dense(p, x, W_l, b8, W_r):
  n, d = x.shape
  bn = 2000
  dn = (((1,), (1,)), ((), ()))

  def body(p_ref, x_ref, wl_ref, b_ref, wr_ref, o_ref):
    agg = p_ref[0] + p_ref[1]
    o_ref[...] = (
        lax.dot_general(agg, wl_ref[...], dn,
                        preferred_element_type=jnp.float32,
                        precision=lax.Precision.HIGHEST)
        + lax.dot_general(x_ref[...], wr_ref[...], dn,
                          preferred_element_type=jnp.float32,
                          precision=lax.Precision.HIGHEST)
        + b_ref[0:1, :])

  return pl.pallas_call(
      body,
      grid=(n // bn,),
      in_specs=[
          pl.BlockSpec((2, bn, d), lambda i: (0, i, 0)),
          pl.BlockSpec((bn, d), lambda i: (i, 0)),
          pl.BlockSpec((d, d), lambda i: (0, 0)),
          pl.BlockSpec((8, d), lambda i: (0, 0)),
          pl.BlockSpec((d, d), lambda i: (0, 0)),
      ],
      out_specs=pl.BlockSpec((bn, d), lambda i: (i, 0)),
      out_shape=jax.ShapeDtypeStruct((n, d), jnp.float32),
  )(p, x, W_l, b8, W_r)


_K = 96


def kernel(x, edge_index, edge_weight, W_l, b_l, W_r):
  n, d = x.shape
  e = edge_weight.shape[0]
  e_per_t = (e + _NW - 1) // _NW
  nb0 = (e_per_t + _K - 1) // _K
  nb = ((nb0 + _IDEPTH - 1) // _IDEPTH) * _IDEPTH
  pad = _NW * nb * _K - e
  row = jnp.pad(edge_index[0].astype(jnp.int32), (0, pad)).reshape(_NW, nb, _K)
  col = jnp.pad(edge_index[1].astype(jnp.int32), (0, pad)).reshape(_NW, nb, _K)
  w2 = jnp.pad(edge_weight, (0, pad)).reshape(_NW, nb, _K)
  p = _make_agg(n, d, nb, _K)(x, row, col, w2)
  b8 = jnp.broadcast_to(b_l.reshape(1, d), (8, d))
  return _dense(p, x, W_l, b8, W_r)


# speedup vs baseline: 1.8618x; 1.8618x over previous
import functools

import jax
import jax.numpy as jnp
from jax import lax
from jax.experimental import pallas as pl
from jax.experimental.pallas import tpu as pltpu
from jax.experimental.pallas import tpu_sc as plsc

_NC = 2
_NS = 16
_NW = _NC * _NS
_L = 16
_H = 64


def _make_agg(n, d, nb, k):
  rows_per_tile = -(-n // (_NS * k)) * k
  n_pad = rows_per_tile * _NS
  nz = rows_per_tile // _H
  mesh = plsc.VectorSubcoreMesh(core_axis_name="c", subcore_axis_name="s")

  @functools.partial(
      pl.kernel,
      out_type=jax.ShapeDtypeStruct((_NC, n_pad, d), jnp.float32),
      mesh=mesh,
      scratch_types=[
          pltpu.VMEM((nb, k), jnp.int32),
          pltpu.VMEM((nb, k), jnp.int32),
          pltpu.VMEM((nb, k), jnp.float32),
          pltpu.VMEM((_H, d), jnp.float32),
          pltpu.VMEM((_H, d), jnp.float32),
          pltpu.VMEM_SHARED((n_pad, d), jnp.float32),
          pltpu.SemaphoreType.DMA,
          pltpu.SemaphoreType.DMA,
      ],
  )
  def agg(x_hbm, row_hbm, col_hbm, w_hbm, out_hbm,
          colv, rowv, wv, r0, r1, acc, g0, g1):
    c = lax.axis_index("c")
    s = lax.axis_index("s")
    wid = c * _NS + s

    pltpu.sync_copy(col_hbm.at[wid], colv)
    pltpu.sync_copy(row_hbm.at[wid], rowv)
    pltpu.sync_copy(w_hbm.at[wid], wv)

    def start_gather(b, off, buf, sem):
      pltpu.async_copy(x_hbm.at[colv.at[b, pl.ds(off, _H)]], buf, sem)

    def wait_gather(b, off, buf, sem):
      pltpu.make_async_copy(
          x_hbm.at[colv.at[b, pl.ds(off, _H)]], buf, sem).wait()

    def scale(buf, b, off):
      def scale16(g, _):
        wvec = wv[b, pl.ds(off + g * _L, _L)]
        for j16 in range(_L):
          w = wvec[j16]
          j = g * _L + j16
          for t in range(d // _L):
            buf[j, pl.ds(t * _L, _L)] = buf[j, pl.ds(t * _L, _L)] * w
        return 0

      lax.fori_loop(0, _H // _L, scale16, 0)

    def scatter(buf, b, off):
      pltpu.sync_copy(buf, acc.at[rowv.at[b, pl.ds(off, _H)]], add=True)

    zero = jnp.zeros((_L,), jnp.float32)

    def zstore(i, _):
      r = i // (d // _L)
      col0 = (i % (d // _L)) * _L
      r0[r, pl.ds(col0, _L)] = zero
      return 0

    lax.fori_loop(0, _H * (d // _L), zstore, 0)
    for t in range(nz):
      pltpu.sync_copy(r0, acc.at[pl.ds(s * rows_per_tile + t * _H, _H)])
    plsc.subcore_barrier()

    start_gather(0, 0, r0, g0)

    def body(b, _):
      start_gather(b, _H, r1, g1)
      wait_gather(b, 0, r0, g0)
      scale(r0, b, 0)
      scatter(r0, b, 0)

      @pl.when(b + 1 < nb)
      def _():
        start_gather(b + 1, 0, r0, g0)

      wait_gather(b, _H, r1, g1)
      scale(r1, b, _H)
      scatter(r1, b, _H)
      return 0

    lax.fori_loop(0, nb, body, 0)

    plsc.subcore_barrier()
    pltpu.sync_copy(acc.at[pl.ds(s * rows_per_tile, rows_per_tile)],
                    out_hbm.at[c, pl.ds(s * rows_per_tile, rows_per_tile)])

  return agg


def _dense(p, x, W_l, b8, W_r):
  n, d = x.shape
  bn = 2000
  dn = (((1,), (1,)), ((), ()))

  def body(p_ref, x_ref, wl_ref, b_ref, wr_ref, o_ref):
    agg = p_ref[0] + p_ref[1]
    o_ref[...] = (
        lax.dot_general(agg, wl_ref[...], dn,
                        preferred_element_type=jnp.float32,
                        precision=lax.Precision.HIGHEST)
        + lax.dot_general(x_ref[...], wr_ref[...], dn,
                          preferred_element_type=jnp.float32,
                          precision=lax.Precision.HIGHEST)
        + b_ref[0:1, :])

  return pl.pallas_call(
      body,
      grid=(n // bn,),
      in_specs=[
          pl.BlockSpec((2, bn, d), lambda i: (0, i, 0)),
          pl.BlockSpec((bn, d), lambda i: (i, 0)),
          pl.BlockSpec((d, d), lambda i: (0, 0)),
          pl.BlockSpec((8, d), lambda i: (0, 0)),
          pl.BlockSpec((d, d), lambda i: (0, 0)),
      ],
      out_specs=pl.BlockSpec((bn, d), lambda i: (i, 0)),
      out_shape=jax.ShapeDtypeStruct((n, d), jnp.float32),
  )(p, x, W_l, b8, W_r)


_K = 128


def kernel(x, edge_index, edge_weight, W_l, b_l, W_r):
  n, d = x.shape
  e = edge_weight.shape[0]
  nb = -(-e // (_NW * _K))
  pad = _NW * nb * _K - e
  row = jnp.pad(edge_index[0].astype(jnp.int32), (0, pad)).reshape(_NW, nb, _K)
  col = jnp.pad(edge_index[1].astype(jnp.int32), (0, pad)).reshape(_NW, nb, _K)
  w2 = jnp.pad(edge_weight, (0, pad)).reshape(_NW, nb, _K)
  p = _make_agg(n, d, nb, _K)(x, row, col, w2)
  b8 = jnp.broadcast_to(b_l.reshape(1, d), (8, d))
  return _dense(p, x, W_l, b8, W_r)
